# Initial kernel scaffold; baseline (speedup 1.0000x reference)
import jax
import jax.numpy as jnp
from jax.experimental import pallas as pl


def kernel(params, surface_type, type_emb, W_pe, b_pe, enc_W1, enc_b1, enc_W2, enc_b2, enc_W3, enc_b3, enc_W4, enc_b4, fsq_Win, fsq_bin, fsq_Wout, fsq_bout, dec_W1, dec_b1, dec_W2, dec_b2, dec_W3, dec_b3, cls_W, cls_b, isc_W, isc_b, decraw_W, decraw_b):
    raise NotImplementedError("write your pallas kernel here")


# speedup vs baseline: 5.3340x; 5.3340x over previous
import functools

import jax
import jax.numpy as jnp
import numpy as np
from jax.experimental import pallas as pl
from jax.experimental.pallas import tpu as pltpu

_LEVELS = np.array([8, 5, 5, 5])
_RAW_DIMS = np.array([7, 9, 10, 11, 12])
_B = 16384
_R = 1024
_NT = 5

_EPS = 1e-3
_HALF_L = ((_LEVELS - 1.0) * (1.0 + _EPS) / 2.0).astype(np.float32)
_OFFSET = np.where(_LEVELS % 2 == 0, 0.5, 0.0).astype(np.float32)
_SHIFT = np.arctanh(_OFFSET / _HALF_L).astype(np.float32)
_HALF_W = (_LEVELS // 2).astype(np.float32)
_BASIS = np.concatenate([[1], np.cumprod(_LEVELS[:-1])]).astype(np.float32)
_MASK_TABLE = (np.arange(12)[None, :] < _RAW_DIMS[:, None]).astype(np.float32)


def _tc_body(stf_ref, params_ref,
             wpeT_ref, bpe_ref,
             w1aT_ref, w1bT_ref, b1_ref,
             w2T_ref, b2_ref, w3T_ref, b3_ref, w4T_ref, b4_ref,
             fwinT_ref, fbin_ref, fwoutT_ref, fbout_ref,
             clsT_ref, clsb_ref, iscT_ref, iscb_ref,
             d1aT_ref, d1bT_ref, db1_ref, d2T_ref, db2_ref, d3T_ref, db3_ref,
             wdrT_ref, bdr_ref, temb_ref,
             shift_ref, halfl_ref, offs_ref, halfw_ref, basis_ref, mtab_ref,
             recon_ref, maskf_ref, cls_ref, isc_ref, zq_ref, idx_ref):
    f32 = jnp.float32
    dot = functools.partial(jnp.dot, preferred_element_type=f32)
    sti = stf_ref[...]
    iota5 = jax.lax.broadcasted_iota(jnp.int32, (_R, _NT), 1)
    onehot = (iota5 == sti).astype(f32)
    emb = dot(onehot, temb_ref[...])

    p5 = dot(params_ref[...], wpeT_ref[...]) + bpe_ref[...]
    pe = onehot[:, 0:1] * p5[:, 0:32]
    for t in range(1, _NT):
        pe = pe + onehot[:, t:t + 1] * p5[:, 32 * t:32 * (t + 1)]

    h = jnp.maximum(dot(pe, w1aT_ref[...]) + dot(emb, w1bT_ref[...]) + b1_ref[...], 0.0)
    h = jnp.maximum(dot(h, w2T_ref[...]) + b2_ref[...], 0.0)
    h = jnp.maximum(dot(h, w3T_ref[...]) + b3_ref[...], 0.0)
    z = dot(h, w4T_ref[...]) + b4_ref[...]

    zp = dot(z, fwinT_ref[...]) + fbin_ref[...]
    bounded = jnp.tanh(zp + shift_ref[...]) * halfl_ref[...] - offs_ref[...]
    rounded = jnp.round(bounded)
    codes = rounded / halfw_ref[...]
    idx_f = jnp.sum((rounded + halfw_ref[...]) * basis_ref[...],
                    axis=1, keepdims=True)
    idx_ref[...] = idx_f.astype(jnp.int32)
    zq = dot(codes, fwoutT_ref[...]) + fbout_ref[...]
    zq_ref[...] = zq

    cls_ref[...] = dot(zq, clsT_ref[...]) + clsb_ref[...]
    isc_ref[...] = dot(zq, iscT_ref[...]) + iscb_ref[...]

    hd = jnp.maximum(dot(zq, d1aT_ref[...]) + dot(emb, d1bT_ref[...]) + db1_ref[...], 0.0)
    hd = jnp.maximum(dot(hd, d2T_ref[...]) + db2_ref[...], 0.0)
    pd = dot(hd, d3T_ref[...]) + db3_ref[...]

    d5 = dot(pd, wdrT_ref[...]) + bdr_ref[...]
    recon = onehot[:, 0:1] * d5[:, 0:12]
    for t in range(1, _NT):
        recon = recon + onehot[:, t:t + 1] * d5[:, 12 * t:12 * (t + 1)]
    recon_ref[...] = recon

    maskf_ref[...] = dot(onehot, mtab_ref[...])


def _full(shape):
    nd = len(shape)
    return pl.BlockSpec(shape, lambda i: (0,) * nd)


def _rows(width, dtype=None):
    return pl.BlockSpec((_R, width), lambda i: (i, 0))


@jax.jit
def _run(stf, params, args):
    grid = _B // _R
    in_specs = [_rows(1), _rows(12)] + [_full(a.shape) for a in args]
    out_shapes = (
        jax.ShapeDtypeStruct((_B, 12), jnp.float32),
        jax.ShapeDtypeStruct((_B, 12), jnp.float32),
        jax.ShapeDtypeStruct((_B, _NT), jnp.float32),
        jax.ShapeDtypeStruct((_B, 2), jnp.float32),
        jax.ShapeDtypeStruct((_B, 128), jnp.float32),
        jax.ShapeDtypeStruct((_B, 1), jnp.int32),
    )
    out_specs = (_rows(12), _rows(12), _rows(_NT), _rows(2), _rows(128), _rows(1))
    return pl.pallas_call(
        _tc_body,
        grid=(grid,),
        in_specs=in_specs,
        out_specs=out_specs,
        out_shape=out_shapes,
        compiler_params=pltpu.CompilerParams(
            dimension_semantics=("arbitrary",),
        ),
    )(stf, params, *args)


def kernel(params, surface_type, type_emb, W_pe, b_pe,
           enc_W1, enc_b1, enc_W2, enc_b2, enc_W3, enc_b3, enc_W4, enc_b4,
           fsq_Win, fsq_bin, fsq_Wout, fsq_bout,
           dec_W1, dec_b1, dec_W2, dec_b2, dec_W3, dec_b3,
           cls_W, cls_b, isc_W, isc_b, decraw_W, decraw_b):
    stf = surface_type.astype(jnp.int32).reshape(_B, 1)
    args = (
        W_pe.reshape(_NT * 32, 12).T,
        b_pe.reshape(1, _NT * 32),
        enc_W1[:, :32].T, enc_W1[:, 32:].T, enc_b1.reshape(1, -1),
        enc_W2.T, enc_b2.reshape(1, -1),
        enc_W3.T, enc_b3.reshape(1, -1),
        enc_W4.T, enc_b4.reshape(1, -1),
        fsq_Win.T, fsq_bin.reshape(1, -1),
        fsq_Wout.T, fsq_bout.reshape(1, -1),
        cls_W.T, cls_b.reshape(1, -1),
        isc_W.T, isc_b.reshape(1, -1),
        dec_W1[:, :128].T, dec_W1[:, 128:].T, dec_b1.reshape(1, -1),
        dec_W2.T, dec_b2.reshape(1, -1),
        dec_W3.T, dec_b3.reshape(1, -1),
        decraw_W.reshape(_NT * 12, 32).T,
        decraw_b.reshape(1, _NT * 12),
        type_emb,
        jnp.asarray(_SHIFT).reshape(1, 4), jnp.asarray(_HALF_L).reshape(1, 4),
        jnp.asarray(_OFFSET).reshape(1, 4), jnp.asarray(_HALF_W).reshape(1, 4),
        jnp.asarray(_BASIS).reshape(1, 4), jnp.asarray(_MASK_TABLE),
    )
    recon, maskf, cls, isc, zq, idx = _run(stf, params, args)
    return recon, maskf > 0.5, cls, isc, zq, idx.reshape(_B)
